# baseline (speedup 1.0000x reference)
import functools

import jax
import jax.numpy as jnp
from jax import lax
from jax.experimental import pallas as pl
from jax.experimental.pallas import tpu as pltpu
from jax.experimental.pallas import tpu_sc as plsc

_NC, _NS = 2, 16
_NW = _NC * _NS


def kernel(arr, index, element):
    M, N = arr.shape
    rpw = M // _NW
    idx16 = jnp.full((16,), index, dtype=jnp.int32)
    elem2d = element.reshape((1, N))
    mesh = plsc.VectorSubcoreMesh(core_axis_name="c", subcore_axis_name="s")

    @functools.partial(
        pl.kernel,
        out_type=jax.ShapeDtypeStruct((M, N), arr.dtype),
        mesh=mesh,
        scratch_types=[pltpu.VMEM((16,), jnp.int32), pltpu.SemaphoreType.DMA],
    )
    def run(arr_hbm, idx_hbm, elem_hbm, out_hbm, idx_v, sem):
        wid = lax.axis_index("s") * _NC + lax.axis_index("c")
        base = wid * rpw
        cp = pltpu.async_copy(
            arr_hbm.at[pl.ds(base, rpw)], out_hbm.at[pl.ds(base, rpw)], sem
        )
        pltpu.sync_copy(idx_hbm, idx_v)
        idx_s = idx_v[...][0]
        cp.wait()

        @pl.when((idx_s >= base) & (idx_s < base + rpw))
        def _():
            pltpu.sync_copy(elem_hbm, out_hbm.at[pl.ds(idx_s, 1)])

    return run(arr, idx16, elem2d)


# speedup vs baseline: 15.9492x; 15.9492x over previous
import functools

import jax
import jax.numpy as jnp
from jax import lax
from jax.experimental import pallas as pl
from jax.experimental.pallas import tpu as pltpu
from jax.experimental.pallas import tpu_sc as plsc

_NC, _NS = 2, 16
_NW = _NC * _NS
_CH = 32


def kernel(arr, index, element):
    M, N = arr.shape
    rpt = M // _NW
    nchunk = rpt // _CH
    idx16 = jnp.full((16,), index, dtype=jnp.int32)
    elem2d = element.reshape((1, N))
    mesh = plsc.VectorSubcoreMesh(core_axis_name="c", subcore_axis_name="s")

    @functools.partial(
        pl.kernel,
        out_type=jax.ShapeDtypeStruct((M, N), arr.dtype),
        mesh=mesh,
        scratch_types=[
            pltpu.VMEM((16,), jnp.int32),
            pltpu.VMEM((_CH, N), jnp.float32),
            pltpu.VMEM((_CH, N), jnp.float32),
            pltpu.SemaphoreType.DMA,
            pltpu.SemaphoreType.DMA,
            pltpu.SemaphoreType.DMA,
            pltpu.SemaphoreType.DMA,
        ],
    )
    def run(arr_hbm, idx_hbm, elem_hbm, out_hbm, idx_v, buf0, buf1,
            gs0, gs1, ss0, ss1):
        wid = lax.axis_index("s") * _NC + lax.axis_index("c")
        base = wid * rpt
        pltpu.sync_copy(idx_hbm, idx_v)
        local = idx_v[...][0] - base
        bufs = (buf0, buf1)
        gsems = (gs0, gs1)
        ssems = (ss0, ss1)

        gathers = [None, None]
        scatters = [None, None]
        for k in range(nchunk + 1):
            if k < nchunk:
                cur = k & 1
                if k >= 2:
                    scatters[cur].wait()
                gathers[cur] = pltpu.async_copy(
                    arr_hbm.at[pl.ds(base + k * _CH, _CH)], bufs[cur],
                    gsems[cur])
            if k >= 1:
                j = k - 1
                cur = j & 1
                gathers[cur].wait()

                @pl.when((local >= j * _CH) & (local < (j + 1) * _CH))
                def _(cur=cur, j=j):
                    pltpu.sync_copy(elem_hbm,
                                    bufs[cur].at[pl.ds(local - j * _CH, 1)])

                scatters[cur] = pltpu.async_copy(
                    bufs[cur], out_hbm.at[pl.ds(base + j * _CH, _CH)],
                    ssems[cur])
        scatters[0].wait()
        scatters[1].wait()

    return run(arr, idx16, elem2d)
